# baseline (speedup 1.0000x reference)
import functools

import jax
import jax.numpy as jnp
from jax import lax
from jax.experimental import pallas as pl
from jax.experimental.pallas import tpu as pltpu
from jax.experimental.pallas import tpu_sc as plsc

HID = 128
N_NODES = 10000
N_PAD = 10240
SC_CORES = 2
SC_SUBCORES = 16
WIN = 128
BLKE = 1024
BLKN = 2048


def _sc_mesh():
    return plsc.VectorSubcoreMesh(core_axis_name="c", subcore_axis_name="s")


def _dot32(a, w):
    return jnp.dot(a, w, preferred_element_type=jnp.float32,
                   precision=lax.Precision.HIGHEST)


def _sc_gather(a, b, col2, row2, ep):
    @functools.partial(
        pl.kernel,
        out_type=(jax.ShapeDtypeStruct((ep, HID), jnp.float32),
                  jax.ShapeDtypeStruct((ep, HID), jnp.float32)),
        mesh=_sc_mesh(),
        scratch_types=[pltpu.SemaphoreType.DMA((2,))],
    )
    def k(a_hbm, b_hbm, c_hbm, r_hbm, oa_hbm, ob_hbm, sem):
        def body(ic, ir, oa_vmem, ob_vmem):
            ca = pltpu.async_copy(a_hbm.at[ic.at[0]], oa_vmem, sem.at[0])
            cb = pltpu.async_copy(b_hbm.at[ir.at[0]], ob_vmem, sem.at[1])
            ca.wait()
            cb.wait()

        pltpu.emit_pipeline(
            body,
            grid=(ep // WIN,),
            in_specs=[
                pl.BlockSpec((1, WIN), lambda i: (0, i)),
                pl.BlockSpec((1, WIN), lambda i: (0, i)),
            ],
            out_specs=[pl.BlockSpec((WIN, HID), lambda i: (i, 0)),
                       pl.BlockSpec((WIN, HID), lambda i: (i, 0))],
            core_axis_name=("c", "s"),
            dimension_semantics=(pltpu.PARALLEL,),
        )(c_hbm, r_hbm, oa_hbm, ob_hbm)

    return k(a, b, col2, row2)


def _sc_scatter(m, col2, zeros_tbl, ep):
    rows = N_PAD // SC_SUBCORES

    hw = WIN // 2

    @functools.partial(
        pl.kernel,
        out_type=jax.ShapeDtypeStruct((SC_CORES, N_PAD, HID), jnp.float32),
        mesh=_sc_mesh(),
        scratch_types=[pltpu.VMEM_SHARED((N_PAD, HID), jnp.float32),
                       pltpu.SemaphoreType.DMA((2,))],
    )
    def k(m_hbm, c_hbm, z_hbm, o_hbm, tbl, sem):
        cid = lax.axis_index("c")
        sid = lax.axis_index("s")
        pltpu.sync_copy(z_hbm.at[pl.ds(sid * rows, rows)],
                        tbl.at[pl.ds(sid * rows, rows)])
        plsc.subcore_barrier()

        def body(m_vmem, ic):
            c0 = pltpu.async_copy(m_vmem.at[pl.ds(0, hw)],
                                  tbl.at[ic.at[0]], sem.at[0], add=True)
            c1 = pltpu.async_copy(m_vmem.at[pl.ds(hw, hw)],
                                  tbl.at[ic.at[1]], sem.at[1], add=True)
            c0.wait()
            c1.wait()

        pltpu.emit_pipeline(
            body,
            grid=(ep // WIN,),
            in_specs=[
                pl.BlockSpec((WIN, HID), lambda i: (i, 0)),
                pl.BlockSpec((2, hw), lambda i: (i, 0)),
            ],
            out_specs=[],
            core_axis_name=("c", "s"),
            dimension_semantics=(pltpu.PARALLEL,),
        )(m_hbm, c_hbm)

        plsc.subcore_barrier()
        pltpu.sync_copy(tbl.at[pl.ds(sid * rows, rows)],
                        o_hbm.at[cid, pl.ds(sid * rows, rows)])

    return k(m, col2, zeros_tbl)


def _tc_edge(ga, gb, eap, we4, w2, b2, w3, b3, ep):
    def body(ga_ref, gb_ref, ea_ref, we4_ref, w2_ref, b2_ref, w3_ref, b3_ref,
             m_ref):
        z = ga_ref[...] + gb_ref[...]
        ea = ea_ref[...]
        we4 = we4_ref[...]
        for j in range(4):
            z = z + ea[:, j:j + 1] * we4[j:j + 1, :]
        z = jnp.maximum(z, 0.0).astype(jnp.bfloat16)
        z2 = jnp.dot(z, w2_ref[...], preferred_element_type=jnp.float32)
        z2 = jnp.maximum(z2 + b2_ref[...], 0.0).astype(jnp.bfloat16)
        m_ref[...] = (jnp.dot(z2, w3_ref[...], preferred_element_type=jnp.float32)
                      + b3_ref[...])

    return pl.pallas_call(
        body,
        grid=(ep // BLKE,),
        in_specs=[
            pl.BlockSpec((BLKE, HID), lambda i: (i, 0)),
            pl.BlockSpec((BLKE, HID), lambda i: (i, 0)),
            pl.BlockSpec((BLKE, 4), lambda i: (i, 0)),
            pl.BlockSpec((4, HID), lambda i: (0, 0)),
            pl.BlockSpec((HID, HID), lambda i: (0, 0)),
            pl.BlockSpec((1, HID), lambda i: (0, 0)),
            pl.BlockSpec((HID, HID), lambda i: (0, 0)),
            pl.BlockSpec((1, HID), lambda i: (0, 0)),
        ],
        out_specs=pl.BlockSpec((BLKE, HID), lambda i: (i, 0)),
        out_shape=jax.ShapeDtypeStruct((ep, HID), jnp.float32),
    )(ga, gb, eap, we4, w2, b2, w3, b3)


def _tc_node(h, p, wn1h, wn1a, bn1, wn2, bn2, wn3, bn3, gamma, beta,
             wa, b1n, wb):
    def body(h_ref, p0_ref, p1_ref, p2_ref, p3_ref, wn1h_ref, wn1a_ref,
             bn1_ref, wn2_ref, bn2_ref, wn3_ref, bn3_ref, g_ref, be_ref,
             wa_ref, b1_ref, wb_ref, ho_ref, ao_ref, bo_ref):
        h0 = h_ref[...]
        aggr = (p0_ref[0] + p1_ref[0]) + (p2_ref[0] + p3_ref[0])
        u = _dot32(h0, wn1h_ref[...])
        u = u + _dot32(aggr, wn1a_ref[...])
        u = jnp.maximum(u + bn1_ref[...], 0.0)
        u = jnp.maximum(_dot32(u, wn2_ref[...]) + bn2_ref[...], 0.0)
        upd = _dot32(u, wn3_ref[...]) + bn3_ref[...]
        v = h0 + upd
        mu = jnp.mean(v, axis=-1, keepdims=True)
        vc = v - mu
        var = jnp.mean(vc * vc, axis=-1, keepdims=True)
        hn = vc * lax.rsqrt(var + 1e-5) * g_ref[...] + be_ref[...]
        ho_ref[...] = hn
        ao_ref[...] = _dot32(hn, wa_ref[...]) + b1_ref[...]
        bo_ref[...] = _dot32(hn, wb_ref[...])

    mat = pl.BlockSpec((HID, HID), lambda i: (0, 0))
    vec = pl.BlockSpec((1, HID), lambda i: (0, 0))
    blk = pl.BlockSpec((BLKN, HID), lambda i: (i, 0))
    pb0 = pl.BlockSpec((1, BLKN, HID), lambda i: (0, i, 0))
    pb1 = pl.BlockSpec((1, BLKN, HID), lambda i: (1, i, 0))
    return pl.pallas_call(
        body,
        grid=(N_PAD // BLKN,),
        in_specs=[
            blk, pb0, pb1, pb0, pb1,
            mat, mat, vec, mat, vec, mat, vec, vec, vec, mat, vec, mat,
        ],
        out_specs=[blk, blk, blk],
        out_shape=[
            jax.ShapeDtypeStruct((N_PAD, HID), jnp.float32),
            jax.ShapeDtypeStruct((N_PAD, HID), jnp.float32),
            jax.ShapeDtypeStruct((N_PAD, HID), jnp.float32),
        ],
    )(h, p[0], p[0], p[1], p[1], wn1h, wn1a, bn1, wn2, bn2, wn3, bn3,
      gamma, beta, wa, b1n, wb)


def _tc_prologue(xp, wp, bp, wa, b1n, wb):
    def body(x_ref, wp_ref, bp_ref, wa_ref, b1_ref, wb_ref,
             ho_ref, ao_ref, bo_ref):
        h = _dot32(x_ref[...], wp_ref[...]) + bp_ref[...]
        ho_ref[...] = h
        ao_ref[...] = _dot32(h, wa_ref[...]) + b1_ref[...]
        bo_ref[...] = _dot32(h, wb_ref[...])

    mat = pl.BlockSpec((HID, HID), lambda i: (0, 0))
    vec = pl.BlockSpec((1, HID), lambda i: (0, 0))
    blk = pl.BlockSpec((BLKN, HID), lambda i: (i, 0))
    return pl.pallas_call(
        body,
        grid=(N_PAD // BLKN,),
        in_specs=[blk, mat, vec, mat, vec, mat],
        out_specs=[blk, blk, blk],
        out_shape=[
            jax.ShapeDtypeStruct((N_PAD, HID), jnp.float32),
            jax.ShapeDtypeStruct((N_PAD, HID), jnp.float32),
            jax.ShapeDtypeStruct((N_PAD, HID), jnp.float32),
        ],
    )(xp, wp, bp, wa, b1n, wb)


def _prep_layer(lp):
    (w1, b1), (w2, b2), (w3, b3) = lp["edge_mlp"]
    (wn1, bn1), (wn2, bn2), (wn3, bn3) = lp["node_mlp"]
    gamma, beta = lp["ln"]
    wpm = w1[2 * HID + 4:2 * HID + 7]
    wa = w1[0:HID].at[3:6].add(wpm)
    wb = w1[HID:2 * HID].at[3:6].add(-wpm)
    bf = jnp.bfloat16
    return {
        "wa": wa, "wb": wb,
        "b1": b1.reshape(1, HID),
        "we4": w1[2 * HID:2 * HID + 4],
        "w2": w2.astype(bf), "b2": b2.reshape(1, HID),
        "w3": w3.astype(bf), "b3": b3.reshape(1, HID),
        "wn1h": wn1[:HID], "wn1a": wn1[HID:],
        "bn1": bn1.reshape(1, HID),
        "wn2": wn2, "bn2": bn2.reshape(1, HID),
        "wn3": wn3, "bn3": bn3.reshape(1, HID),
        "gamma": gamma.reshape(1, HID), "beta": beta.reshape(1, HID),
    }


def kernel(x, pos, edge_attr, params, edge_index):
    e = edge_index.shape[1]
    win_tot = WIN * SC_CORES * SC_SUBCORES * 2
    ep = ((e + win_tot - 1) // win_tot) * win_tot
    pad_e = ep - e
    row = edge_index[0]
    col = edge_index[1]
    colp = jnp.concatenate([col, jnp.full((pad_e,), N_NODES, jnp.int32)])
    rowp = jnp.concatenate([row, jnp.zeros((pad_e,), jnp.int32)])
    col2 = colp.reshape(1, ep)
    row2 = rowp.reshape(1, ep)
    eap = jnp.concatenate(
        [edge_attr, jnp.zeros((pad_e, 4), jnp.float32)], axis=0)
    xp = jnp.pad(x, ((0, N_PAD - x.shape[0]), (0, 0)))
    zeros_tbl = jnp.zeros((N_PAD, HID), jnp.float32)

    prepped = [_prep_layer(lp) for lp in params["layers"]]
    wp, bp = params["input_proj"]
    h, a, b = _tc_prologue(xp, wp, bp.reshape(1, HID),
                           prepped[0]["wa"], prepped[0]["b1"],
                           prepped[0]["wb"])
    n_layers = len(prepped)
    half = ep // 2
    cols = (col2[:, :half], col2[:, half:])
    rows_i = (row2[:, :half], row2[:, half:])
    colsc = colp.reshape(ep // (WIN // 2), WIN // 2)
    colscs = (colsc[:half // (WIN // 2)], colsc[half // (WIN // 2):])
    eaps = (eap[:half], eap[half:])
    for l in range(n_layers):
        pr = prepped[l]
        ga1, gb1 = _sc_gather(a, b, cols[0], rows_i[0], half)
        ga2, gb2 = _sc_gather(a, b, cols[1], rows_i[1], half)
        m1 = _tc_edge(ga1, gb1, eaps[0], pr["we4"], pr["w2"], pr["b2"],
                      pr["w3"], pr["b3"], half)
        p1 = _sc_scatter(m1, colscs[0], zeros_tbl, half)
        m2 = _tc_edge(ga2, gb2, eaps[1], pr["we4"], pr["w2"], pr["b2"],
                      pr["w3"], pr["b3"], half)
        p2 = _sc_scatter(m2, colscs[1], zeros_tbl, half)
        nxt = prepped[(l + 1) % n_layers]
        h, a, b = _tc_node(h, (p1, p2), pr["wn1h"], pr["wn1a"], pr["bn1"],
                           pr["wn2"], pr["bn2"], pr["wn3"], pr["bn3"],
                           pr["gamma"], pr["beta"],
                           nxt["wa"], nxt["b1"], nxt["wb"])
    return h[:N_NODES]


# speedup vs baseline: 1.0414x; 1.0414x over previous
import functools

import jax
import jax.numpy as jnp
from jax import lax
from jax.experimental import pallas as pl
from jax.experimental.pallas import tpu as pltpu
from jax.experimental.pallas import tpu_sc as plsc

HID = 128
N_NODES = 10000
N_PAD = 10240
SC_CORES = 2
SC_SUBCORES = 16
WIN = 128
BLKE = 1024
BLKN = 2048


def _sc_mesh():
    return plsc.VectorSubcoreMesh(core_axis_name="c", subcore_axis_name="s")


def _dot32(a, w):
    return jnp.dot(a, w, preferred_element_type=jnp.float32,
                   precision=lax.Precision.HIGHEST)


def _sc_gather(a, b, col2, row2, ep):
    @functools.partial(
        pl.kernel,
        out_type=(jax.ShapeDtypeStruct((ep, HID), jnp.float32),
                  jax.ShapeDtypeStruct((ep, HID), jnp.float32)),
        mesh=_sc_mesh(),
        scratch_types=[pltpu.SemaphoreType.DMA((2,))],
    )
    def k(a_hbm, b_hbm, c_hbm, r_hbm, oa_hbm, ob_hbm, sem):
        def body(ic, ir, oa_vmem, ob_vmem):
            ca = pltpu.async_copy(a_hbm.at[ic.at[0]], oa_vmem, sem.at[0])
            cb = pltpu.async_copy(b_hbm.at[ir.at[0]], ob_vmem, sem.at[1])
            ca.wait()
            cb.wait()

        pltpu.emit_pipeline(
            body,
            grid=(ep // WIN,),
            in_specs=[
                pl.BlockSpec((1, WIN), lambda i: (0, i)),
                pl.BlockSpec((1, WIN), lambda i: (0, i)),
            ],
            out_specs=[pl.BlockSpec((WIN, HID), lambda i: (i, 0)),
                       pl.BlockSpec((WIN, HID), lambda i: (i, 0))],
            core_axis_name=("c", "s"),
            dimension_semantics=(pltpu.PARALLEL,),
        )(c_hbm, r_hbm, oa_hbm, ob_hbm)

    return k(a, b, col2, row2)


def _sc_scatter(m, col2, zeros_tbl, ep):
    rows = N_PAD // SC_SUBCORES

    @functools.partial(
        pl.kernel,
        out_type=jax.ShapeDtypeStruct((SC_CORES, N_PAD, HID), jnp.float32),
        mesh=_sc_mesh(),
        scratch_types=[pltpu.VMEM_SHARED((N_PAD, HID), jnp.float32)],
    )
    def k(m_hbm, c_hbm, z_hbm, o_hbm, tbl):
        cid = lax.axis_index("c")
        sid = lax.axis_index("s")
        pltpu.sync_copy(z_hbm.at[pl.ds(sid * rows, rows)],
                        tbl.at[pl.ds(sid * rows, rows)])
        plsc.subcore_barrier()

        def body(m_vmem, ic):
            pltpu.sync_copy(m_vmem, tbl.at[ic.at[0]], add=True)

        pltpu.emit_pipeline(
            body,
            grid=(ep // WIN,),
            in_specs=[
                pl.BlockSpec((WIN, HID), lambda i: (i, 0)),
                pl.BlockSpec((1, WIN), lambda i: (0, i)),
            ],
            out_specs=[],
            core_axis_name=("c", "s"),
            dimension_semantics=(pltpu.PARALLEL,),
        )(m_hbm, c_hbm)

        plsc.subcore_barrier()
        pltpu.sync_copy(tbl.at[pl.ds(sid * rows, rows)],
                        o_hbm.at[cid, pl.ds(sid * rows, rows)])

    return k(m, col2, zeros_tbl)


def _tc_edge(ga, gb, eap, we4, w2, b2, w3, b3, ep):
    def body(ga_ref, gb_ref, ea_ref, we4_ref, w2_ref, b2_ref, w3_ref, b3_ref,
             m_ref):
        z = ga_ref[...] + gb_ref[...]
        ea = ea_ref[...]
        we4 = we4_ref[...]
        for j in range(4):
            z = z + ea[:, j:j + 1] * we4[j:j + 1, :]
        z = jnp.maximum(z, 0.0).astype(jnp.bfloat16)
        z2 = jnp.dot(z, w2_ref[...], preferred_element_type=jnp.float32)
        z2 = jnp.maximum(z2 + b2_ref[...], 0.0).astype(jnp.bfloat16)
        m_ref[...] = (jnp.dot(z2, w3_ref[...], preferred_element_type=jnp.float32)
                      + b3_ref[...])

    return pl.pallas_call(
        body,
        grid=(ep // BLKE,),
        in_specs=[
            pl.BlockSpec((BLKE, HID), lambda i: (i, 0)),
            pl.BlockSpec((BLKE, HID), lambda i: (i, 0)),
            pl.BlockSpec((BLKE, 4), lambda i: (i, 0)),
            pl.BlockSpec((4, HID), lambda i: (0, 0)),
            pl.BlockSpec((HID, HID), lambda i: (0, 0)),
            pl.BlockSpec((1, HID), lambda i: (0, 0)),
            pl.BlockSpec((HID, HID), lambda i: (0, 0)),
            pl.BlockSpec((1, HID), lambda i: (0, 0)),
        ],
        out_specs=pl.BlockSpec((BLKE, HID), lambda i: (i, 0)),
        out_shape=jax.ShapeDtypeStruct((ep, HID), jnp.float32),
    )(ga, gb, eap, we4, w2, b2, w3, b3)


def _tc_node(h, p, wn1h, wn1a, bn1, wn2, bn2, wn3, bn3, gamma, beta,
             wa, b1n, wb):
    def body(h_ref, p0_ref, p1_ref, p2_ref, p3_ref, wn1h_ref, wn1a_ref,
             bn1_ref, wn2_ref, bn2_ref, wn3_ref, bn3_ref, g_ref, be_ref,
             wa_ref, b1_ref, wb_ref, ho_ref, ao_ref, bo_ref):
        h0 = h_ref[...]
        aggr = (p0_ref[0] + p1_ref[0]) + (p2_ref[0] + p3_ref[0])
        u = _dot32(h0, wn1h_ref[...])
        u = u + _dot32(aggr, wn1a_ref[...])
        u = jnp.maximum(u + bn1_ref[...], 0.0)
        u = jnp.maximum(_dot32(u, wn2_ref[...]) + bn2_ref[...], 0.0)
        upd = _dot32(u, wn3_ref[...]) + bn3_ref[...]
        v = h0 + upd
        mu = jnp.mean(v, axis=-1, keepdims=True)
        vc = v - mu
        var = jnp.mean(vc * vc, axis=-1, keepdims=True)
        hn = vc * lax.rsqrt(var + 1e-5) * g_ref[...] + be_ref[...]
        ho_ref[...] = hn
        ao_ref[...] = _dot32(hn, wa_ref[...]) + b1_ref[...]
        bo_ref[...] = _dot32(hn, wb_ref[...])

    mat = pl.BlockSpec((HID, HID), lambda i: (0, 0))
    vec = pl.BlockSpec((1, HID), lambda i: (0, 0))
    blk = pl.BlockSpec((BLKN, HID), lambda i: (i, 0))
    pb0 = pl.BlockSpec((1, BLKN, HID), lambda i: (0, i, 0))
    pb1 = pl.BlockSpec((1, BLKN, HID), lambda i: (1, i, 0))
    return pl.pallas_call(
        body,
        grid=(N_PAD // BLKN,),
        in_specs=[
            blk, pb0, pb1, pb0, pb1,
            mat, mat, vec, mat, vec, mat, vec, vec, vec, mat, vec, mat,
        ],
        out_specs=[blk, blk, blk],
        out_shape=[
            jax.ShapeDtypeStruct((N_PAD, HID), jnp.float32),
            jax.ShapeDtypeStruct((N_PAD, HID), jnp.float32),
            jax.ShapeDtypeStruct((N_PAD, HID), jnp.float32),
        ],
    )(h, p[0], p[0], p[1], p[1], wn1h, wn1a, bn1, wn2, bn2, wn3, bn3,
      gamma, beta, wa, b1n, wb)


def _tc_prologue(xp, wp, bp, wa, b1n, wb):
    def body(x_ref, wp_ref, bp_ref, wa_ref, b1_ref, wb_ref,
             ho_ref, ao_ref, bo_ref):
        h = _dot32(x_ref[...], wp_ref[...]) + bp_ref[...]
        ho_ref[...] = h
        ao_ref[...] = _dot32(h, wa_ref[...]) + b1_ref[...]
        bo_ref[...] = _dot32(h, wb_ref[...])

    mat = pl.BlockSpec((HID, HID), lambda i: (0, 0))
    vec = pl.BlockSpec((1, HID), lambda i: (0, 0))
    blk = pl.BlockSpec((BLKN, HID), lambda i: (i, 0))
    return pl.pallas_call(
        body,
        grid=(N_PAD // BLKN,),
        in_specs=[blk, mat, vec, mat, vec, mat],
        out_specs=[blk, blk, blk],
        out_shape=[
            jax.ShapeDtypeStruct((N_PAD, HID), jnp.float32),
            jax.ShapeDtypeStruct((N_PAD, HID), jnp.float32),
            jax.ShapeDtypeStruct((N_PAD, HID), jnp.float32),
        ],
    )(xp, wp, bp, wa, b1n, wb)


def _prep_layer(lp):
    (w1, b1), (w2, b2), (w3, b3) = lp["edge_mlp"]
    (wn1, bn1), (wn2, bn2), (wn3, bn3) = lp["node_mlp"]
    gamma, beta = lp["ln"]
    wpm = w1[2 * HID + 4:2 * HID + 7]
    wa = w1[0:HID].at[3:6].add(wpm)
    wb = w1[HID:2 * HID].at[3:6].add(-wpm)
    bf = jnp.bfloat16
    return {
        "wa": wa, "wb": wb,
        "b1": b1.reshape(1, HID),
        "we4": w1[2 * HID:2 * HID + 4],
        "w2": w2.astype(bf), "b2": b2.reshape(1, HID),
        "w3": w3.astype(bf), "b3": b3.reshape(1, HID),
        "wn1h": wn1[:HID], "wn1a": wn1[HID:],
        "bn1": bn1.reshape(1, HID),
        "wn2": wn2, "bn2": bn2.reshape(1, HID),
        "wn3": wn3, "bn3": bn3.reshape(1, HID),
        "gamma": gamma.reshape(1, HID), "beta": beta.reshape(1, HID),
    }


def kernel(x, pos, edge_attr, params, edge_index):
    e = edge_index.shape[1]
    win_tot = WIN * SC_CORES * SC_SUBCORES * 2
    ep = ((e + win_tot - 1) // win_tot) * win_tot
    pad_e = ep - e
    row = edge_index[0]
    col = edge_index[1]
    colp = jnp.concatenate([col, jnp.full((pad_e,), N_NODES, jnp.int32)])
    rowp = jnp.concatenate([row, jnp.zeros((pad_e,), jnp.int32)])
    col2 = colp.reshape(1, ep)
    row2 = rowp.reshape(1, ep)
    eap = jnp.concatenate(
        [edge_attr, jnp.zeros((pad_e, 4), jnp.float32)], axis=0)
    xp = jnp.pad(x, ((0, N_PAD - x.shape[0]), (0, 0)))
    zeros_tbl = jnp.zeros((N_PAD, HID), jnp.float32)

    prepped = [_prep_layer(lp) for lp in params["layers"]]
    wp, bp = params["input_proj"]
    h, a, b = _tc_prologue(xp, wp, bp.reshape(1, HID),
                           prepped[0]["wa"], prepped[0]["b1"],
                           prepped[0]["wb"])
    n_layers = len(prepped)
    half = ep // 2
    cols = (col2[:, :half], col2[:, half:])
    rows_i = (row2[:, :half], row2[:, half:])
    eaps = (eap[:half], eap[half:])
    for l in range(n_layers):
        pr = prepped[l]
        ga1, gb1 = _sc_gather(a, b, cols[0], rows_i[0], half)
        ga2, gb2 = _sc_gather(a, b, cols[1], rows_i[1], half)
        m1 = _tc_edge(ga1, gb1, eaps[0], pr["we4"], pr["w2"], pr["b2"],
                      pr["w3"], pr["b3"], half)
        p1 = _sc_scatter(m1, cols[0], zeros_tbl, half)
        m2 = _tc_edge(ga2, gb2, eaps[1], pr["we4"], pr["w2"], pr["b2"],
                      pr["w3"], pr["b3"], half)
        p2 = _sc_scatter(m2, cols[1], zeros_tbl, half)
        nxt = prepped[(l + 1) % n_layers]
        h, a, b = _tc_node(h, (p1, p2), pr["wn1h"], pr["wn1a"], pr["bn1"],
                           pr["wn2"], pr["bn2"], pr["wn3"], pr["bn3"],
                           pr["gamma"], pr["beta"],
                           nxt["wa"], nxt["b1"], nxt["wb"])
    return h[:N_NODES]
